# baseline (speedup 1.0000x reference)
import functools

import jax
import jax.numpy as jnp
from jax import lax
from jax.experimental import pallas as pl
from jax.experimental.pallas import tpu as pltpu
from jax.experimental.pallas import tpu_sc as plsc

_B = 4
_F = 128
_STEP = 2
_OUTF = _F // _STEP
_NROWS = _B * _OUTF
_NC, _NS = 2, 16
_NW = _NC * _NS
_RPW = _NROWS // _NW


def _make_sc_copy():
    mesh = plsc.VectorSubcoreMesh(
        core_axis_name="c", subcore_axis_name="s",
        num_cores=_NC, num_subcores=_NS)

    @functools.partial(
        pl.kernel,
        out_type=(
            jax.ShapeDtypeStruct((_B, _OUTF, 3, 112, 112), jnp.float32),
            jax.ShapeDtypeStruct((_B, _OUTF, 1024), jnp.float32),
        ),
        mesh=mesh,
        scratch_types=[
            pltpu.VMEM((4, 3, 56, 112), jnp.float32),
            pltpu.VMEM((_RPW, 1024), jnp.float32),
            pltpu.SemaphoreType.DMA((4,)),
            pltpu.SemaphoreType.DMA((4,)),
            pltpu.SemaphoreType.DMA,
            pltpu.SemaphoreType.DMA,
        ],
    )
    def sc_copy(vin, ain, vout, aout, vbuf, abuf, insems, outsems, asem_in, asem_out):
        wid = lax.axis_index("s") * _NC + lax.axis_index("c")
        base = wid * _RPW
        b = base // _OUTF
        i0 = base % _OUTF

        a_in = [
            pltpu.make_async_copy(ain.at[b, _STEP * (i0 + j)], abuf.at[j], asem_in)
            for j in range(_RPW)
        ]
        _NM = 2 * _RPW
        v_in = [
            pltpu.make_async_copy(
                vin.at[b, _STEP * (i0 + m // 2), :, pl.ds(56 * (m % 2), 56), :],
                vbuf.at[m % 4], insems.at[m % 4])
            for m in range(_NM)
        ]
        v_out = [
            pltpu.make_async_copy(
                vbuf.at[m % 4],
                vout.at[b, i0 + m // 2, :, pl.ds(56 * (m % 2), 56), :],
                outsems.at[m % 4])
            for m in range(_NM)
        ]
        a_out = pltpu.make_async_copy(abuf, aout.at[b, pl.ds(i0, _RPW)], asem_out)

        for c in a_in:
            c.start()
        for m in range(4):
            v_in[m].start()
        for m in range(_NM):
            if m >= 1:
                v_out[m - 1].wait()
                if m + 3 < _NM:
                    v_in[m + 3].start()
            v_in[m].wait()
            v_out[m].start()
        v_out[_NM - 1].wait()
        for c in a_in:
            c.wait()
        a_out.start()
        a_out.wait()

    return sc_copy


_sc_copy = _make_sc_copy()


def kernel(video, audio):
    return _sc_copy(video, audio)
